# baseline (speedup 1.0000x reference)
import functools

import jax
import jax.numpy as jnp
import numpy as np
from jax import lax
from jax.experimental import pallas as pl
from jax.experimental.pallas import tpu as pltpu
from jax.experimental.pallas import tpu_sc as plsc

N_LEVELS = 12
TABLE_SIZE = 1048576
MASK = TABLE_SIZE - 1
FEATURES = 2
N_POINTS = 131072
ALPHA_SCALE = 0.1

PI2 = np.int32(2654435761 - (1 << 32))
PI3 = np.int32(805459861)

_LEVELS = (4.0 * 2.0 ** (0.43 * np.arange(N_LEVELS, dtype=np.float64))).astype(np.float32)

NC, NS = 2, 16
NW = NC * NS
PPW = N_POINTS // NW
CHUNK = 32
NGROUP = CHUNK // 16
NCHUNK = PPW // CHUNK

N_DENSE = 7
_D = [int(np.ceil(float(_LEVELS[l]))) + 1 for l in range(N_DENSE)]
_DCNT = [d * d * d for d in _D]
_DBASE = [2 * int(np.sum(_DCNT[:l])) for l in range(N_DENSE)]
ND = int(np.sum(_DCNT))

N_HBM = N_LEVELS - N_DENSE
GPB = N_HBM * 8 * CHUNK

_KPT0 = -(-ND // NS)
BROUNDS = -(-_KPT0 // GPB)
BE = -(-_KPT0 // (BROUNDS * 16)) * 16
KPT = BE * BROUNDS
NDP = KPT * NS


def _hash_rows_np():
    rows = np.zeros((NDP, 2), np.int32)
    cols = np.zeros((NDP,), np.int32)
    e = 0
    for l in range(N_DENSE):
        d = _D[l]
        cz, cy, cx = np.meshgrid(np.arange(d), np.arange(d), np.arange(d),
                                 indexing='ij')
        cx = (cx + l).astype(np.int64).reshape(-1)
        cy = (cy + l).astype(np.int64).reshape(-1)
        cz = (cz + l).astype(np.int64).reshape(-1)
        h = ((cx ^ (cy * 2654435761) ^ (cz * 805459861)) % TABLE_SIZE).astype(np.int64)
        off0 = l * (2 * TABLE_SIZE) + (h >> 7) * 256 + (h & 127)
        r0 = (off0 >> 3).astype(np.int32)
        n = d * d * d
        rows[e:e + n, 0] = r0
        rows[e:e + n, 1] = r0 + 16
        cols[e:e + n] = (h & 7).astype(np.int32)
        e += n
    return rows.reshape(-1), cols


_DROWS_NP, _DCOLS_NP = _hash_rows_np()


FB = 256


def _sc_body(xt_hbm, grid_hbm, drows_hbm, dcols_hbm, out_hbm,
             x_v, idxr_va, idxc_va, vals_va,
             idxr_vb, idxc_vb, vals_vb,
             feats_v, dense_v, shared_v, semA, semB):
    cid = lax.axis_index("c")
    sid = lax.axis_index("s")
    wid = sid * NC + cid
    base = wid * PPW
    iota1 = lax.iota(jnp.int32, 16)
    iota2 = iota1 * 2

    dbase0 = sid * (2 * KPT)
    for r in range(BROUNDS):
        eo = r * BE
        pltpu.sync_copy(drows_hbm.at[pl.ds(sid * (2 * KPT) + 2 * eo, 2 * BE)],
                        idxr_va.at[pl.ds(0, 2 * BE)])
        pltpu.sync_copy(dcols_hbm.at[pl.ds(sid * KPT + eo, BE)],
                        idxc_va.at[pl.ds(0, BE)])
        pltpu.async_copy(grid_hbm.at[idxr_va.at[pl.ds(0, 2 * BE)]],
                         vals_va.at[pl.ds(0, 2 * BE)], semA).wait()

        def build_fn(b, c2, eo=eo):
            cols = idxc_va[pl.ds(b * 16, 16)]
            rows = iota2 + b * 32
            v0 = plsc.load_gather(vals_va, [rows, cols])
            v1 = plsc.load_gather(vals_va, [rows + 1, cols])
            e0 = iota2 + (dbase0 + 2 * eo + b * 32)
            plsc.store_scatter(dense_v, [e0], v0)
            plsc.store_scatter(dense_v, [e0 + 1], v1)
            return c2

        lax.fori_loop(0, BE // 16, build_fn, 0)
    pltpu.sync_copy(dense_v.at[pl.ds(dbase0, 2 * KPT)],
                    shared_v.at[pl.ds(dbase0, 2 * KPT)])
    plsc.subcore_barrier()
    pltpu.sync_copy(shared_v, dense_v)

    pltpu.sync_copy(xt_hbm.at[:, pl.ds(base, PPW)], x_v)

    def do_hash(ci, idxr_v, idxc_v, vals_v, sem):
        cb = ci * CHUNK

        def hash_group(g, c2):
            off = cb + g * 16
            px = x_v[0, pl.ds(off, 16)]
            py = x_v[1, pl.ds(off, 16)]
            pz = x_v[2, pl.ds(off, 16)]
            gb = g * (N_HBM * 8 * 16)
            for l in range(N_DENSE, N_LEVELS):
                s = float(_LEVELS[l])
                fl = float(l)
                xs = px * s + fl
                ys = py * s + fl
                zs = pz * s + fl
                ix = xs.astype(jnp.int32)
                iy = ys.astype(jnp.int32)
                iz = zs.astype(jnp.int32)
                hx0 = ix
                hx1 = ix + 1
                by0 = iy * PI2
                by1 = by0 + PI2
                cz0 = iz * PI3
                cz1 = cz0 + PI3
                t00 = hx0 ^ by0
                t01 = hx0 ^ by1
                t10 = hx1 ^ by0
                t11 = hx1 ^ by1
                lbase = l * (2 * TABLE_SIZE)
                corners = (t00 ^ cz0, t00 ^ cz1, t01 ^ cz0, t01 ^ cz1,
                           t10 ^ cz0, t10 ^ cz1, t11 ^ cz0, t11 ^ cz1)
                for k in range(8):
                    q = corners[k] & MASK
                    off0 = (q + q) - (q & 127) + lbase
                    r0 = off0 >> 3
                    slot = gb + ((l - N_DENSE) * 8 + k) * 16
                    idxr_v[pl.ds(slot, 16)] = r0
                    idxr_v[pl.ds(GPB + slot, 16)] = r0 + 16
                    idxc_v[pl.ds(slot, 16)] = q & 7
            return c2

        lax.fori_loop(0, NGROUP, hash_group, 0)
        pltpu.async_copy(grid_hbm.at[idxr_v], vals_v, sem)

    def do_weight(ci, idxc_v, vals_v):
        cb = ci * CHUNK
        fcb = (ci % (FB // CHUNK)) * CHUNK

        def weight_group(g, c2):
            off = cb + g * 16
            px = x_v[0, pl.ds(off, 16)]
            py = x_v[1, pl.ds(off, 16)]
            pz = x_v[2, pl.ds(off, 16)]
            gb = g * (N_HBM * 8 * 16)
            for l in range(N_LEVELS):
                s = float(_LEVELS[l])
                fl = float(l)
                xs = px * s + fl
                ys = py * s + fl
                zs = pz * s + fl
                ix = xs.astype(jnp.int32)
                iy = ys.astype(jnp.int32)
                iz = zs.astype(jnp.int32)
                fx = xs - ix.astype(jnp.float32)
                fy = ys - iy.astype(jnp.float32)
                fz = zs - iz.astype(jnp.float32)
                gx = 1.0 - fx
                gy = 1.0 - fy
                gz = 1.0 - fz
                wxy = (gx * gy, gx * fy, fx * gy, fx * fy)
                acc0 = None
                acc1 = None
                if l < N_DENSE:
                    d = _D[l]
                    e000 = (ix + iy * d + iz * (d * d)) * 2 + (
                        _DBASE[l] - 2 * l * (1 + d + d * d))
                    for k in range(8):
                        i, j, kz = k >> 2, (k >> 1) & 1, k & 1
                        w = wxy[k // 2] * (fz if kz else gz)
                        e = e000 + (i * 2 + j * (2 * d) + kz * (2 * d * d))
                        v0 = plsc.load_gather(dense_v, [e])
                        v1 = plsc.load_gather(dense_v, [e + 1])
                        if acc0 is None:
                            acc0 = w * v0
                            acc1 = w * v1
                        else:
                            acc0 = acc0 + w * v0
                            acc1 = acc1 + w * v1
                else:
                    rowb = gb + (l - N_DENSE) * 128
                    for k in range(8):
                        w = wxy[k // 2] * (fz if (k & 1) else gz)
                        slot = rowb + k * 16
                        rows = iota1 + slot
                        colv = idxc_v[pl.ds(slot, 16)]
                        v0 = plsc.load_gather(vals_v, [rows, colv])
                        v1 = plsc.load_gather(vals_v, [rows + GPB, colv])
                        if acc0 is None:
                            acc0 = w * v0
                            acc1 = w * v1
                        else:
                            acc0 = acc0 + w * v0
                            acc1 = acc1 + w * v1
                feats_v[2 * l, pl.ds(fcb + g * 16, 16)] = acc0
                feats_v[2 * l + 1, pl.ds(fcb + g * 16, 16)] = acc1
            return c2

        lax.fori_loop(0, NGROUP, weight_group, 0)

    bufA = (idxr_va, idxc_va, vals_va, semA)
    bufB = (idxr_vb, idxc_vb, vals_vb, semB)

    def wait_gather(buf):
        idxr_v, idxc_v, vals_v, sem = buf
        pltpu.make_async_copy(grid_hbm.at[idxr_v], vals_v, sem).wait()

    do_hash(0, *bufA)
    do_hash(1, *bufB)
    PAIRS_PER_FLUSH = FB // (2 * CHUNK)

    def pair_fn(p, carry):
        ci = 2 * p

        wait_gather(bufA)
        do_weight(ci, bufA[1], bufA[2])

        @pl.when(p + 1 < NCHUNK // 2)
        def _():
            do_hash(ci + 2, *bufA)

        wait_gather(bufB)
        do_weight(ci + 1, bufB[1], bufB[2])

        @pl.when(p + 1 < NCHUNK // 2)
        def _():
            do_hash(ci + 3, *bufB)

        @pl.when(p % PAIRS_PER_FLUSH == PAIRS_PER_FLUSH - 1)
        def _():
            fb = (p // PAIRS_PER_FLUSH) * FB
            pltpu.sync_copy(feats_v, out_hbm.at[:, pl.ds(base + fb, FB)])

        return carry

    lax.fori_loop(0, NCHUNK // 2, pair_fn, 0)


@functools.lru_cache(maxsize=None)
def _build_sc_encode():
    return pl.kernel(
        _sc_body,
        out_type=jax.ShapeDtypeStruct((2 * N_LEVELS, N_POINTS), jnp.float32),
        mesh=plsc.VectorSubcoreMesh(core_axis_name="c", subcore_axis_name="s",
                                    num_cores=NC, num_subcores=NS),
        scratch_types=[
            pltpu.VMEM((3, PPW), jnp.float32),
            pltpu.VMEM((2 * GPB,), jnp.int32),
            pltpu.VMEM((GPB,), jnp.int32),
            pltpu.VMEM((2 * GPB, 8), jnp.float32),
            pltpu.VMEM((2 * GPB,), jnp.int32),
            pltpu.VMEM((GPB,), jnp.int32),
            pltpu.VMEM((2 * GPB, 8), jnp.float32),
            pltpu.VMEM((2 * N_LEVELS, FB), jnp.float32),
            pltpu.VMEM((2 * NDP,), jnp.float32),
            pltpu.VMEM_SHARED((2 * NDP,), jnp.float32),
            pltpu.SemaphoreType.DMA,
            pltpu.SemaphoreType.DMA,
        ],
        compiler_params=pltpu.CompilerParams(
            needs_layout_passes=False,
            use_tc_tiling_on_sc=False,
        ),
    )


BN = 4096


def _mlp_body(f_ref, w1_ref, b1_ref, w2_ref, b2_ref, w3_ref, b3_ref, o_ref):
    f = f_ref[...]
    h = lax.dot_general(w1_ref[...], f, (((1,), (0,)), ((), ())),
                        preferred_element_type=jnp.float32) + b1_ref[...]
    h = jnp.where(h >= 0, h, 0.01 * h)
    h = lax.dot_general(w2_ref[...], h, (((1,), (0,)), ((), ())),
                        preferred_element_type=jnp.float32) + b2_ref[...]
    h = jnp.where(h >= 0, h, 0.01 * h)
    h = lax.dot_general(w3_ref[...], h, (((1,), (0,)), ((), ())),
                        preferred_element_type=jnp.float32) + b3_ref[...]
    sigma = h[0:1]
    alpha = jnp.minimum(h[1:2], 0.0) * ALPHA_SCALE
    o_ref[...] = jnp.concatenate([sigma, alpha], axis=0)


def _mlp(feats, w1t, b1, w2t, b2, w3t, b3):
    d_in = 2 * N_LEVELS
    grid_n = N_POINTS // BN
    return pl.pallas_call(
        _mlp_body,
        grid=(grid_n,),
        in_specs=[
            pl.BlockSpec((d_in, BN), lambda j: (0, j)),
            pl.BlockSpec((64, d_in), lambda j: (0, 0)),
            pl.BlockSpec((64, 1), lambda j: (0, 0)),
            pl.BlockSpec((32, 64), lambda j: (0, 0)),
            pl.BlockSpec((32, 1), lambda j: (0, 0)),
            pl.BlockSpec((2, 32), lambda j: (0, 0)),
            pl.BlockSpec((2, 1), lambda j: (0, 0)),
        ],
        out_specs=pl.BlockSpec((2, BN), lambda j: (0, j)),
        out_shape=jax.ShapeDtypeStruct((2, N_POINTS), jnp.float32),
    )(feats, w1t, b1, w2t, b2, w3t, b3)


def kernel(x, grid, W1, b1, W2, b2, W3, b3):
    xt = x.T
    gridf = (grid.reshape(N_LEVELS, TABLE_SIZE // 128, 128, FEATURES)
             .transpose(0, 1, 3, 2)
             .reshape(N_LEVELS * TABLE_SIZE * FEATURES // 8, 8))
    feats = _build_sc_encode()(xt, gridf, jnp.asarray(_DROWS_NP),
                               jnp.asarray(_DCOLS_NP))
    out = _mlp(feats, W1.T, b1.reshape(64, 1), W2.T, b2.reshape(32, 1),
               W3.T, b3.reshape(2, 1))
    return out.T


# speedup vs baseline: 1.0106x; 1.0106x over previous
import functools

import jax
import jax.numpy as jnp
import numpy as np
from jax import lax
from jax.experimental import pallas as pl
from jax.experimental.pallas import tpu as pltpu
from jax.experimental.pallas import tpu_sc as plsc

N_LEVELS = 12
TABLE_SIZE = 1048576
MASK = TABLE_SIZE - 1
FEATURES = 2
N_POINTS = 131072
ALPHA_SCALE = 0.1

PI2 = np.int32(2654435761 - (1 << 32))
PI3 = np.int32(805459861)

_LEVELS = (4.0 * 2.0 ** (0.43 * np.arange(N_LEVELS, dtype=np.float64))).astype(np.float32)

NC, NS = 2, 16
NW = NC * NS
PPW = N_POINTS // NW
CHUNK = 16
NGROUP = CHUNK // 16
NCHUNK = PPW // CHUNK
NBUF = 4

N_DENSE = 7
_D = [int(np.ceil(float(_LEVELS[l]))) + 1 for l in range(N_DENSE)]
_DCNT = [d * d * d for d in _D]
_DBASE = [2 * int(np.sum(_DCNT[:l])) for l in range(N_DENSE)]
ND = int(np.sum(_DCNT))

N_HBM = N_LEVELS - N_DENSE
GPB = N_HBM * 8 * CHUNK

_KPT0 = -(-ND // NS)
BROUNDS = -(-_KPT0 // GPB)
BE = -(-_KPT0 // (BROUNDS * 16)) * 16
KPT = BE * BROUNDS
NDP = KPT * NS


def _hash_rows_np():
    rows = np.zeros((NDP, 2), np.int32)
    cols = np.zeros((NDP,), np.int32)
    e = 0
    for l in range(N_DENSE):
        d = _D[l]
        cz, cy, cx = np.meshgrid(np.arange(d), np.arange(d), np.arange(d),
                                 indexing='ij')
        cx = (cx + l).astype(np.int64).reshape(-1)
        cy = (cy + l).astype(np.int64).reshape(-1)
        cz = (cz + l).astype(np.int64).reshape(-1)
        h = ((cx ^ (cy * 2654435761) ^ (cz * 805459861)) % TABLE_SIZE).astype(np.int64)
        off0 = l * (2 * TABLE_SIZE) + (h >> 7) * 256 + (h & 127)
        r0 = (off0 >> 3).astype(np.int32)
        n = d * d * d
        rows[e:e + n, 0] = r0
        rows[e:e + n, 1] = r0 + 16
        cols[e:e + n] = (h & 7).astype(np.int32)
        e += n
    return rows.reshape(-1), cols


_DROWS_NP, _DCOLS_NP = _hash_rows_np()


FB = 128


def _sc_body(xt_hbm, grid_hbm, drows_hbm, dcols_hbm, out_hbm,
             x_v,
             idxr_v0, idxc_v0, vals_v0, sem0,
             idxr_v1, idxc_v1, vals_v1, sem1,
             idxr_v2, idxc_v2, vals_v2, sem2,
             idxr_v3, idxc_v3, vals_v3, sem3,
             feats_v, dense_v, shared_v):
    idxr_va, idxc_va, vals_va, semA = idxr_v0, idxc_v0, vals_v0, sem0
    cid = lax.axis_index("c")
    sid = lax.axis_index("s")
    wid = sid * NC + cid
    base = wid * PPW
    iota1 = lax.iota(jnp.int32, 16)
    iota2 = iota1 * 2

    dbase0 = sid * (2 * KPT)
    for r in range(BROUNDS):
        eo = r * BE
        pltpu.sync_copy(drows_hbm.at[pl.ds(sid * (2 * KPT) + 2 * eo, 2 * BE)],
                        idxr_va.at[pl.ds(0, 2 * BE)])
        pltpu.sync_copy(dcols_hbm.at[pl.ds(sid * KPT + eo, BE)],
                        idxc_va.at[pl.ds(0, BE)])
        pltpu.async_copy(grid_hbm.at[idxr_va.at[pl.ds(0, 2 * BE)]],
                         vals_va.at[pl.ds(0, 2 * BE)], semA).wait()

        def build_fn(b, c2, eo=eo):
            cols = idxc_va[pl.ds(b * 16, 16)]
            rows = iota2 + b * 32
            v0 = plsc.load_gather(vals_va, [rows, cols])
            v1 = plsc.load_gather(vals_va, [rows + 1, cols])
            e0 = iota2 + (dbase0 + 2 * eo + b * 32)
            plsc.store_scatter(dense_v, [e0], v0)
            plsc.store_scatter(dense_v, [e0 + 1], v1)
            return c2

        lax.fori_loop(0, BE // 16, build_fn, 0)
    pltpu.sync_copy(dense_v.at[pl.ds(dbase0, 2 * KPT)],
                    shared_v.at[pl.ds(dbase0, 2 * KPT)])
    plsc.subcore_barrier()
    pltpu.sync_copy(shared_v, dense_v)

    pltpu.sync_copy(xt_hbm.at[:, pl.ds(base, PPW)], x_v)

    def do_hash(ci, idxr_v, idxc_v, vals_v, sem):
        cb = ci * CHUNK

        def hash_group(g, c2):
            off = cb + g * 16
            px = x_v[0, pl.ds(off, 16)]
            py = x_v[1, pl.ds(off, 16)]
            pz = x_v[2, pl.ds(off, 16)]
            gb = g * (N_HBM * 8 * 16)
            for l in range(N_DENSE, N_LEVELS):
                s = float(_LEVELS[l])
                fl = float(l)
                xs = px * s + fl
                ys = py * s + fl
                zs = pz * s + fl
                ix = xs.astype(jnp.int32)
                iy = ys.astype(jnp.int32)
                iz = zs.astype(jnp.int32)
                hx0 = ix
                hx1 = ix + 1
                by0 = iy * PI2
                by1 = by0 + PI2
                cz0 = iz * PI3
                cz1 = cz0 + PI3
                t00 = hx0 ^ by0
                t01 = hx0 ^ by1
                t10 = hx1 ^ by0
                t11 = hx1 ^ by1
                lbase = l * (2 * TABLE_SIZE)
                corners = (t00 ^ cz0, t00 ^ cz1, t01 ^ cz0, t01 ^ cz1,
                           t10 ^ cz0, t10 ^ cz1, t11 ^ cz0, t11 ^ cz1)
                for k in range(8):
                    q = corners[k] & MASK
                    off0 = (q + q) - (q & 127) + lbase
                    r0 = off0 >> 3
                    slot = gb + ((l - N_DENSE) * 8 + k) * 16
                    idxr_v[pl.ds(slot, 16)] = r0
                    idxr_v[pl.ds(GPB + slot, 16)] = r0 + 16
                    idxc_v[pl.ds(slot, 16)] = q & 7
            return c2

        lax.fori_loop(0, NGROUP, hash_group, 0)
        pltpu.async_copy(grid_hbm.at[idxr_v], vals_v, sem)

    def do_weight(ci, idxc_v, vals_v):
        cb = ci * CHUNK
        fcb = (ci % (FB // CHUNK)) * CHUNK

        def weight_group(g, c2):
            off = cb + g * 16
            px = x_v[0, pl.ds(off, 16)]
            py = x_v[1, pl.ds(off, 16)]
            pz = x_v[2, pl.ds(off, 16)]
            gb = g * (N_HBM * 8 * 16)
            for l in range(N_LEVELS):
                s = float(_LEVELS[l])
                fl = float(l)
                xs = px * s + fl
                ys = py * s + fl
                zs = pz * s + fl
                ix = xs.astype(jnp.int32)
                iy = ys.astype(jnp.int32)
                iz = zs.astype(jnp.int32)
                fx = xs - ix.astype(jnp.float32)
                fy = ys - iy.astype(jnp.float32)
                fz = zs - iz.astype(jnp.float32)
                gx = 1.0 - fx
                gy = 1.0 - fy
                gz = 1.0 - fz
                wxy = (gx * gy, gx * fy, fx * gy, fx * fy)
                acc0 = None
                acc1 = None
                if l < N_DENSE:
                    d = _D[l]
                    e000 = (ix + iy * d + iz * (d * d)) * 2 + (
                        _DBASE[l] - 2 * l * (1 + d + d * d))
                    for k in range(8):
                        i, j, kz = k >> 2, (k >> 1) & 1, k & 1
                        w = wxy[k // 2] * (fz if kz else gz)
                        e = e000 + (i * 2 + j * (2 * d) + kz * (2 * d * d))
                        v0 = plsc.load_gather(dense_v, [e])
                        v1 = plsc.load_gather(dense_v, [e + 1])
                        if acc0 is None:
                            acc0 = w * v0
                            acc1 = w * v1
                        else:
                            acc0 = acc0 + w * v0
                            acc1 = acc1 + w * v1
                else:
                    rowb = gb + (l - N_DENSE) * 128
                    for k in range(8):
                        w = wxy[k // 2] * (fz if (k & 1) else gz)
                        slot = rowb + k * 16
                        rows = iota1 + slot
                        colv = idxc_v[pl.ds(slot, 16)]
                        v0 = plsc.load_gather(vals_v, [rows, colv])
                        v1 = plsc.load_gather(vals_v, [rows + GPB, colv])
                        if acc0 is None:
                            acc0 = w * v0
                            acc1 = w * v1
                        else:
                            acc0 = acc0 + w * v0
                            acc1 = acc1 + w * v1
                feats_v[2 * l, pl.ds(fcb + g * 16, 16)] = acc0
                feats_v[2 * l + 1, pl.ds(fcb + g * 16, 16)] = acc1
            return c2

        lax.fori_loop(0, NGROUP, weight_group, 0)

    bufs = [(idxr_v0, idxc_v0, vals_v0, sem0),
            (idxr_v1, idxc_v1, vals_v1, sem1),
            (idxr_v2, idxc_v2, vals_v2, sem2),
            (idxr_v3, idxc_v3, vals_v3, sem3)]

    def wait_gather(buf):
        idxr_v, idxc_v, vals_v, sem = buf
        pltpu.make_async_copy(grid_hbm.at[idxr_v], vals_v, sem).wait()

    do_hash(0, *bufs[0])
    do_hash(1, *bufs[1])
    do_hash(2, *bufs[2])
    CPF = FB // CHUNK

    def quad_fn(t, carry):
        ci0 = 4 * t
        for j in range(NBUF):
            ci = ci0 + j
            wait_gather(bufs[j])

            @pl.when(ci + 3 < NCHUNK)
            def _(j=j, ci=ci):
                do_hash(ci + 3, *bufs[(j + 3) % NBUF])

            do_weight(ci, bufs[j][1], bufs[j][2])

            @pl.when(ci % CPF == CPF - 1)
            def _(ci=ci):
                fb = (ci // CPF) * FB
                pltpu.sync_copy(feats_v, out_hbm.at[:, pl.ds(base + fb, FB)])

        return carry

    lax.fori_loop(0, NCHUNK // NBUF, quad_fn, 0)


@functools.lru_cache(maxsize=None)
def _build_sc_encode():
    return pl.kernel(
        _sc_body,
        out_type=jax.ShapeDtypeStruct((2 * N_LEVELS, N_POINTS), jnp.float32),
        mesh=plsc.VectorSubcoreMesh(core_axis_name="c", subcore_axis_name="s",
                                    num_cores=NC, num_subcores=NS),
        scratch_types=[
            pltpu.VMEM((3, PPW), jnp.float32),
        ] + [
            t
            for _ in range(NBUF)
            for t in (pltpu.VMEM((2 * GPB,), jnp.int32),
                      pltpu.VMEM((GPB,), jnp.int32),
                      pltpu.VMEM((2 * GPB, 8), jnp.float32),
                      pltpu.SemaphoreType.DMA)
        ] + [
            pltpu.VMEM((2 * N_LEVELS, FB), jnp.float32),
            pltpu.VMEM((2 * NDP,), jnp.float32),
            pltpu.VMEM_SHARED((2 * NDP,), jnp.float32),
        ],
        compiler_params=pltpu.CompilerParams(
            needs_layout_passes=False,
            use_tc_tiling_on_sc=False,
        ),
    )


BN = 4096


def _mlp_body(f_ref, w1_ref, b1_ref, w2_ref, b2_ref, w3_ref, b3_ref, o_ref):
    f = f_ref[...]
    h = lax.dot_general(w1_ref[...], f, (((1,), (0,)), ((), ())),
                        preferred_element_type=jnp.float32) + b1_ref[...]
    h = jnp.where(h >= 0, h, 0.01 * h)
    h = lax.dot_general(w2_ref[...], h, (((1,), (0,)), ((), ())),
                        preferred_element_type=jnp.float32) + b2_ref[...]
    h = jnp.where(h >= 0, h, 0.01 * h)
    h = lax.dot_general(w3_ref[...], h, (((1,), (0,)), ((), ())),
                        preferred_element_type=jnp.float32) + b3_ref[...]
    sigma = h[0:1]
    alpha = jnp.minimum(h[1:2], 0.0) * ALPHA_SCALE
    o_ref[...] = jnp.concatenate([sigma, alpha], axis=0)


def _mlp(feats, w1t, b1, w2t, b2, w3t, b3):
    d_in = 2 * N_LEVELS
    grid_n = N_POINTS // BN
    return pl.pallas_call(
        _mlp_body,
        grid=(grid_n,),
        in_specs=[
            pl.BlockSpec((d_in, BN), lambda j: (0, j)),
            pl.BlockSpec((64, d_in), lambda j: (0, 0)),
            pl.BlockSpec((64, 1), lambda j: (0, 0)),
            pl.BlockSpec((32, 64), lambda j: (0, 0)),
            pl.BlockSpec((32, 1), lambda j: (0, 0)),
            pl.BlockSpec((2, 32), lambda j: (0, 0)),
            pl.BlockSpec((2, 1), lambda j: (0, 0)),
        ],
        out_specs=pl.BlockSpec((2, BN), lambda j: (0, j)),
        out_shape=jax.ShapeDtypeStruct((2, N_POINTS), jnp.float32),
    )(feats, w1t, b1, w2t, b2, w3t, b3)


def kernel(x, grid, W1, b1, W2, b2, W3, b3):
    xt = x.T
    gridf = (grid.reshape(N_LEVELS, TABLE_SIZE // 128, 128, FEATURES)
             .transpose(0, 1, 3, 2)
             .reshape(N_LEVELS * TABLE_SIZE * FEATURES // 8, 8))
    feats = _build_sc_encode()(xt, gridf, jnp.asarray(_DROWS_NP),
                               jnp.asarray(_DCOLS_NP))
    out = _mlp(feats, W1.T, b1.reshape(64, 1), W2.T, b2.reshape(32, 1),
               W3.T, b3.reshape(2, 1))
    return out.T
